# baseline (speedup 1.0000x reference)
import jax
import jax.numpy as jnp
from jax import lax
from jax.experimental import pallas as pl
from jax.experimental.pallas import tpu as pltpu
from jax.experimental.pallas import tpu_sc as plsc

SEQ = 200
DIM = 128
BATCH = 1024
NC = 2
NS = 16
NW = NC * NS
B_TOTAL = BATCH * SEQ
B_PER_W = B_TOTAL // NW
CHUNK = SEQ
N_CHUNKS = B_PER_W // CHUNK
LANES = 16
VECS_PER_ROW = DIM // LANES
PART_SIZES = (8, 16, 176)


def _sc_body(idx_hbm, table_hbm, pos_hbm, out_hbm,
             idx_v, pos_v, buf0, buf1, buf2,
             gsem0, gsem1, gsem2, ssem0, ssem1, ssem2, psem):
  wid = lax.axis_index("s") * NC + lax.axis_index("c")
  base = wid * B_PER_W

  POS_HEAD = 64
  pltpu.sync_copy(idx_hbm.at[pl.ds(base, B_PER_W)], idx_v)
  pos_head = pltpu.async_copy(
      pos_hbm.at[pl.ds(0, POS_HEAD)], pos_v.at[pl.ds(0, POS_HEAD)], psem)

  bufs = (buf0, buf1, buf2)
  gsems = (gsem0, gsem1, gsem2)
  ssems = (ssem0, ssem1, ssem2)

  def gather_desc(g, b):
    off = pl.multiple_of(g * CHUNK, 8)
    return pltpu.make_async_copy(
        table_hbm.at[idx_v.at[pl.ds(off, CHUNK)]], bufs[b], gsems[b])

  def chunk_scatter_desc(g, b):
    off = pl.multiple_of(base + g * CHUNK, 8)
    return pltpu.make_async_copy(
        bufs[b], out_hbm.at[pl.ds(off, CHUNK)], ssems[b])

  def add_part(b, r0, rows):
    @plsc.parallel_loop(r0, r0 + rows)
    def _row(r):
      for k in range(VECS_PER_ROW):
        sl = pl.ds(k * LANES, LANES)
        plsc.addupdate(bufs[b].at[r, sl], pos_v[r, sl])

  def part_scatter(g, b, r0, rows):
    off = pl.multiple_of(base + g * CHUNK + r0, 8)
    pltpu.async_copy(
        bufs[b].at[pl.ds(r0, rows)],
        out_hbm.at[pl.ds(off, rows)], ssems[b])

  def process_chunk(g, b, issue_g=None, issue_b=None, drain_g=None,
                    drain_b=None):
    gather_desc(g, b).wait()
    r0 = 0
    for i, rows in enumerate(PART_SIZES):
      add_part(b, r0, rows)
      part_scatter(g, b, r0, rows)
      r0 += rows
      if i == 0:
        if drain_g is not None:
          chunk_scatter_desc(drain_g, drain_b).wait()
        if issue_g is not None:
          gather_desc(issue_g, issue_b).start()

  gather_desc(0, 0).start()
  pos_tail = pltpu.async_copy(
      pos_hbm.at[pl.ds(POS_HEAD, SEQ - POS_HEAD)],
      pos_v.at[pl.ds(POS_HEAD, SEQ - POS_HEAD)], psem)
  gather_desc(1, 1).start()
  pos_head.wait()
  gather_desc(0, 0).wait()
  r0 = 0
  for i, rows in enumerate(PART_SIZES):
    if r0 + rows > POS_HEAD and r0 <= POS_HEAD:
      pos_tail.wait()
    add_part(0, r0, rows)
    part_scatter(0, 0, r0, rows)
    r0 += rows
    if i == 0:
      gather_desc(2, 2).start()
  process_chunk(1, 1, issue_g=3, issue_b=0, drain_g=0, drain_b=0)

  def triple_body(q, carry):
    for j in range(3):
      g = 3 * q + 2 + j
      b = (2 + j) % 3
      recycle_b = (1 + j) % 3
      process_chunk(g, b, issue_g=g + 2, issue_b=recycle_b,
                    drain_g=g - 1, drain_b=recycle_b)
    return carry
  lax.fori_loop(0, (N_CHUNKS - 5) // 3, triple_body, 0)

  process_chunk(N_CHUNKS - 3, (N_CHUNKS - 3) % 3,
                issue_g=N_CHUNKS - 1, issue_b=(N_CHUNKS - 1) % 3,
                drain_g=N_CHUNKS - 4, drain_b=(N_CHUNKS - 4) % 3)
  process_chunk(N_CHUNKS - 2, (N_CHUNKS - 2) % 3)
  process_chunk(N_CHUNKS - 1, (N_CHUNKS - 1) % 3)
  for g in (N_CHUNKS - 3, N_CHUNKS - 2, N_CHUNKS - 1):
    chunk_scatter_desc(g, g % 3).wait()


@jax.jit
def _run(idx_flat, table, pos):
  kern = pl.kernel(
      _sc_body,
      out_type=jax.ShapeDtypeStruct((B_TOTAL, DIM), jnp.float32),
      mesh=plsc.VectorSubcoreMesh(
          core_axis_name="c", subcore_axis_name="s",
          num_cores=NC, num_subcores=NS),
      scratch_types=[
          pltpu.VMEM((B_PER_W,), jnp.int32),
          pltpu.VMEM((SEQ, DIM), jnp.float32),
          pltpu.VMEM((CHUNK, DIM), jnp.float32),
          pltpu.VMEM((CHUNK, DIM), jnp.float32),
          pltpu.VMEM((CHUNK, DIM), jnp.float32),
          pltpu.SemaphoreType.DMA,
          pltpu.SemaphoreType.DMA,
          pltpu.SemaphoreType.DMA,
          pltpu.SemaphoreType.DMA,
          pltpu.SemaphoreType.DMA,
          pltpu.SemaphoreType.DMA,
          pltpu.SemaphoreType.DMA,
      ],
  )
  return kern(idx_flat, table, pos)


def kernel(inputs, input_embedding_matrix, position_embedding_matrix):
  idx_flat = inputs.reshape(B_TOTAL)
  out = _run(idx_flat, input_embedding_matrix, position_embedding_matrix)
  return out.reshape(BATCH, SEQ, DIM)


# speedup vs baseline: 1.1016x; 1.1016x over previous
import jax
import jax.numpy as jnp
from jax import lax
from jax.experimental import pallas as pl
from jax.experimental.pallas import tpu as pltpu
from jax.experimental.pallas import tpu_sc as plsc

SEQ = 200
DIM = 128
BATCH = 1024
NC = 2
NS = 16
NW = NC * NS
B_TOTAL = BATCH * SEQ
B_PER_W = B_TOTAL // NW
CHUNK = SEQ
N_CHUNKS = B_PER_W // CHUNK
LANES = 16
VECS_PER_ROW = DIM // LANES
PART_SIZES = (8, 8, 16, 32, 64, 72)


def _sc_body(idx_hbm, table_hbm, pos_hbm, out_hbm,
             idx_v, pos_v, buf0, buf1, buf2,
             gsem0, gsem1, gsem2, ssem0, ssem1, ssem2, psem):
  wid = lax.axis_index("s") * NC + lax.axis_index("c")
  base = wid * B_PER_W

  POS_HEAD = 64
  pltpu.sync_copy(idx_hbm.at[pl.ds(base, B_PER_W)], idx_v)
  pos_head = pltpu.async_copy(
      pos_hbm.at[pl.ds(0, POS_HEAD)], pos_v.at[pl.ds(0, POS_HEAD)], psem)

  bufs = (buf0, buf1, buf2)
  gsems = (gsem0, gsem1, gsem2)
  ssems = (ssem0, ssem1, ssem2)

  def gather_desc(g, b):
    off = pl.multiple_of(g * CHUNK, 8)
    return pltpu.make_async_copy(
        table_hbm.at[idx_v.at[pl.ds(off, CHUNK)]], bufs[b], gsems[b])

  def chunk_scatter_desc(g, b):
    off = pl.multiple_of(base + g * CHUNK, 8)
    return pltpu.make_async_copy(
        bufs[b], out_hbm.at[pl.ds(off, CHUNK)], ssems[b])

  def add_part(b, r0, rows):
    @plsc.parallel_loop(r0, r0 + rows)
    def _row(r):
      for k in range(VECS_PER_ROW):
        sl = pl.ds(k * LANES, LANES)
        plsc.addupdate(bufs[b].at[r, sl], pos_v[r, sl])

  def part_scatter(g, b, r0, rows):
    off = pl.multiple_of(base + g * CHUNK + r0, 8)
    pltpu.async_copy(
        bufs[b].at[pl.ds(r0, rows)],
        out_hbm.at[pl.ds(off, rows)], ssems[b])

  def process_chunk(g, b, issue_g=None, issue_b=None, drain_g=None,
                    drain_b=None):
    gather_desc(g, b).wait()
    r0 = 0
    for i, rows in enumerate(PART_SIZES):
      add_part(b, r0, rows)
      part_scatter(g, b, r0, rows)
      r0 += rows
      if i == 0:
        if drain_g is not None:
          chunk_scatter_desc(drain_g, drain_b).wait()
        if issue_g is not None:
          gather_desc(issue_g, issue_b).start()

  gather_desc(0, 0).start()
  pos_tail = pltpu.async_copy(
      pos_hbm.at[pl.ds(POS_HEAD, SEQ - POS_HEAD)],
      pos_v.at[pl.ds(POS_HEAD, SEQ - POS_HEAD)], psem)
  gather_desc(1, 1).start()
  pos_head.wait()
  gather_desc(0, 0).wait()
  r0 = 0
  for i, rows in enumerate(PART_SIZES):
    if r0 + rows > POS_HEAD and r0 <= POS_HEAD:
      pos_tail.wait()
    add_part(0, r0, rows)
    part_scatter(0, 0, r0, rows)
    r0 += rows
    if i == 0:
      gather_desc(2, 2).start()
  process_chunk(1, 1, issue_g=3, issue_b=0, drain_g=0, drain_b=0)

  def triple_body(q, carry):
    for j in range(3):
      g = 3 * q + 2 + j
      b = (2 + j) % 3
      recycle_b = (1 + j) % 3
      process_chunk(g, b, issue_g=g + 2, issue_b=recycle_b,
                    drain_g=g - 1, drain_b=recycle_b)
    return carry
  lax.fori_loop(0, (N_CHUNKS - 5) // 3, triple_body, 0)

  process_chunk(N_CHUNKS - 3, (N_CHUNKS - 3) % 3,
                issue_g=N_CHUNKS - 1, issue_b=(N_CHUNKS - 1) % 3,
                drain_g=N_CHUNKS - 4, drain_b=(N_CHUNKS - 4) % 3)
  process_chunk(N_CHUNKS - 2, (N_CHUNKS - 2) % 3)
  process_chunk(N_CHUNKS - 1, (N_CHUNKS - 1) % 3)
  for g in (N_CHUNKS - 3, N_CHUNKS - 2, N_CHUNKS - 1):
    chunk_scatter_desc(g, g % 3).wait()


@jax.jit
def _run(idx_flat, table, pos):
  kern = pl.kernel(
      _sc_body,
      out_type=jax.ShapeDtypeStruct((B_TOTAL, DIM), jnp.float32),
      mesh=plsc.VectorSubcoreMesh(
          core_axis_name="c", subcore_axis_name="s",
          num_cores=NC, num_subcores=NS),
      scratch_types=[
          pltpu.VMEM((B_PER_W,), jnp.int32),
          pltpu.VMEM((SEQ, DIM), jnp.float32),
          pltpu.VMEM((CHUNK, DIM), jnp.float32),
          pltpu.VMEM((CHUNK, DIM), jnp.float32),
          pltpu.VMEM((CHUNK, DIM), jnp.float32),
          pltpu.SemaphoreType.DMA,
          pltpu.SemaphoreType.DMA,
          pltpu.SemaphoreType.DMA,
          pltpu.SemaphoreType.DMA,
          pltpu.SemaphoreType.DMA,
          pltpu.SemaphoreType.DMA,
          pltpu.SemaphoreType.DMA,
      ],
  )
  return kern(idx_flat, table, pos)


def kernel(inputs, input_embedding_matrix, position_embedding_matrix):
  idx_flat = inputs.reshape(B_TOTAL)
  out = _run(idx_flat, input_embedding_matrix, position_embedding_matrix)
  return out.reshape(BATCH, SEQ, DIM)


# speedup vs baseline: 1.1104x; 1.0079x over previous
import jax
import jax.numpy as jnp
from jax import lax
from jax.experimental import pallas as pl
from jax.experimental.pallas import tpu as pltpu
from jax.experimental.pallas import tpu_sc as plsc

SEQ = 200
DIM = 128
BATCH = 1024
NC = 2
NS = 16
NW = NC * NS
B_TOTAL = BATCH * SEQ
B_PER_W = B_TOTAL // NW
CHUNK = SEQ
N_CHUNKS = B_PER_W // CHUNK
LANES = 16
VECS_PER_ROW = DIM // LANES
PART_SIZES = (8, 16, 32, 64, 80)


def _sc_body(idx_hbm, table_hbm, pos_hbm, out_hbm,
             idx_v, pos_v, buf0, buf1, buf2,
             gsem0, gsem1, gsem2, ssem0, ssem1, ssem2, psem):
  wid = lax.axis_index("s") * NC + lax.axis_index("c")
  base = wid * B_PER_W

  POS_HEAD = 64
  pltpu.sync_copy(idx_hbm.at[pl.ds(base, B_PER_W)], idx_v)
  pos_head = pltpu.async_copy(
      pos_hbm.at[pl.ds(0, POS_HEAD)], pos_v.at[pl.ds(0, POS_HEAD)], psem)

  bufs = (buf0, buf1, buf2)
  gsems = (gsem0, gsem1, gsem2)
  ssems = (ssem0, ssem1, ssem2)

  def gather_desc(g, b):
    off = pl.multiple_of(g * CHUNK, 8)
    return pltpu.make_async_copy(
        table_hbm.at[idx_v.at[pl.ds(off, CHUNK)]], bufs[b], gsems[b])

  def chunk_scatter_desc(g, b):
    off = pl.multiple_of(base + g * CHUNK, 8)
    return pltpu.make_async_copy(
        bufs[b], out_hbm.at[pl.ds(off, CHUNK)], ssems[b])

  def add_part(b, r0, rows):
    @plsc.parallel_loop(r0, r0 + rows)
    def _row(r):
      for k in range(VECS_PER_ROW):
        sl = pl.ds(k * LANES, LANES)
        plsc.addupdate(bufs[b].at[r, sl], pos_v[r, sl])

  def part_scatter(g, b, r0, rows):
    off = pl.multiple_of(base + g * CHUNK + r0, 8)
    pltpu.async_copy(
        bufs[b].at[pl.ds(r0, rows)],
        out_hbm.at[pl.ds(off, rows)], ssems[b])

  def process_chunk(g, b, issue_g=None, issue_b=None, drain_g=None,
                    drain_b=None):
    gather_desc(g, b).wait()
    if drain_g is not None:
      chunk_scatter_desc(drain_g, drain_b).wait()
    if issue_g is not None:
      gather_desc(issue_g, issue_b).start()
    r0 = 0
    for rows in PART_SIZES:
      add_part(b, r0, rows)
      part_scatter(g, b, r0, rows)
      r0 += rows

  gather_desc(0, 0).start()
  pos_tail = pltpu.async_copy(
      pos_hbm.at[pl.ds(POS_HEAD, SEQ - POS_HEAD)],
      pos_v.at[pl.ds(POS_HEAD, SEQ - POS_HEAD)], psem)
  gather_desc(1, 1).start()
  pos_head.wait()
  gather_desc(0, 0).wait()
  r0 = 0
  for i, rows in enumerate(PART_SIZES):
    if r0 + rows > POS_HEAD and r0 <= POS_HEAD:
      pos_tail.wait()
    add_part(0, r0, rows)
    part_scatter(0, 0, r0, rows)
    r0 += rows
    if i == 0:
      gather_desc(2, 2).start()
  process_chunk(1, 1, issue_g=3, issue_b=0, drain_g=0, drain_b=0)

  def triple_body(q, carry):
    for j in range(3):
      g = 3 * q + 2 + j
      b = (2 + j) % 3
      recycle_b = (1 + j) % 3
      process_chunk(g, b, issue_g=g + 2, issue_b=recycle_b,
                    drain_g=g - 1, drain_b=recycle_b)
    return carry
  lax.fori_loop(0, (N_CHUNKS - 5) // 3, triple_body, 0)

  process_chunk(N_CHUNKS - 3, (N_CHUNKS - 3) % 3,
                issue_g=N_CHUNKS - 1, issue_b=(N_CHUNKS - 1) % 3,
                drain_g=N_CHUNKS - 4, drain_b=(N_CHUNKS - 4) % 3)
  process_chunk(N_CHUNKS - 2, (N_CHUNKS - 2) % 3)
  process_chunk(N_CHUNKS - 1, (N_CHUNKS - 1) % 3)
  for g in (N_CHUNKS - 3, N_CHUNKS - 2, N_CHUNKS - 1):
    chunk_scatter_desc(g, g % 3).wait()


@jax.jit
def _run(idx_flat, table, pos):
  kern = pl.kernel(
      _sc_body,
      out_type=jax.ShapeDtypeStruct((B_TOTAL, DIM), jnp.float32),
      mesh=plsc.VectorSubcoreMesh(
          core_axis_name="c", subcore_axis_name="s",
          num_cores=NC, num_subcores=NS),
      scratch_types=[
          pltpu.VMEM((B_PER_W,), jnp.int32),
          pltpu.VMEM((SEQ, DIM), jnp.float32),
          pltpu.VMEM((CHUNK, DIM), jnp.float32),
          pltpu.VMEM((CHUNK, DIM), jnp.float32),
          pltpu.VMEM((CHUNK, DIM), jnp.float32),
          pltpu.SemaphoreType.DMA,
          pltpu.SemaphoreType.DMA,
          pltpu.SemaphoreType.DMA,
          pltpu.SemaphoreType.DMA,
          pltpu.SemaphoreType.DMA,
          pltpu.SemaphoreType.DMA,
          pltpu.SemaphoreType.DMA,
      ],
  )
  return kern(idx_flat, table, pos)


def kernel(inputs, input_embedding_matrix, position_embedding_matrix):
  idx_flat = inputs.reshape(B_TOTAL)
  out = _run(idx_flat, input_embedding_matrix, position_embedding_matrix)
  return out.reshape(BATCH, SEQ, DIM)


# speedup vs baseline: 1.1202x; 1.0089x over previous
import jax
import jax.numpy as jnp
from jax import lax
from jax.experimental import pallas as pl
from jax.experimental.pallas import tpu as pltpu
from jax.experimental.pallas import tpu_sc as plsc

SEQ = 200
DIM = 128
BATCH = 1024
NC = 2
NS = 16
NW = NC * NS
B_TOTAL = BATCH * SEQ
B_PER_W = B_TOTAL // NW
CHUNK = SEQ
N_CHUNKS = B_PER_W // CHUNK
LANES = 16
VECS_PER_ROW = DIM // LANES
PART_SIZES = (8, 16, 32, 64, 80)


def _sc_body(idx_hbm, table_hbm, pos_hbm, out_hbm,
             idx_v, pos_v, buf0, buf1, buf2,
             gsem0, gsem1, gsem2, ssem0, ssem1, ssem2, psem):
  wid = lax.axis_index("s") * NC + lax.axis_index("c")
  base = wid * B_PER_W

  POS_HEAD = 64
  pltpu.sync_copy(idx_hbm.at[pl.ds(base, B_PER_W)], idx_v)
  pos_head = pltpu.async_copy(
      pos_hbm.at[pl.ds(0, POS_HEAD)], pos_v.at[pl.ds(0, POS_HEAD)], psem)

  bufs = (buf0, buf1, buf2)
  gsems = (gsem0, gsem1, gsem2)
  ssems = (ssem0, ssem1, ssem2)

  def gather_desc(g, b):
    off = pl.multiple_of(g * CHUNK, 8)
    return pltpu.make_async_copy(
        table_hbm.at[idx_v.at[pl.ds(off, CHUNK)]], bufs[b], gsems[b])

  def chunk_scatter_desc(g, b):
    off = pl.multiple_of(base + g * CHUNK, 8)
    return pltpu.make_async_copy(
        bufs[b], out_hbm.at[pl.ds(off, CHUNK)], ssems[b])

  def add_part(b, r0, rows):
    @plsc.parallel_loop(r0, r0 + rows)
    def _row(r):
      for k in range(VECS_PER_ROW):
        sl = pl.ds(k * LANES, LANES)
        plsc.addupdate(bufs[b].at[r, sl], pos_v[r, sl])

  def part_scatter(g, b, r0, rows):
    off = pl.multiple_of(base + g * CHUNK + r0, 8)
    pltpu.async_copy(
        bufs[b].at[pl.ds(r0, rows)],
        out_hbm.at[pl.ds(off, rows)], ssems[b])

  def process_chunk(g, b, issue_g=None, issue_b=None, drain_g=None,
                    drain_b=None):
    gather_desc(g, b).wait()
    r0 = 0
    for i, rows in enumerate(PART_SIZES):
      add_part(b, r0, rows)
      part_scatter(g, b, r0, rows)
      r0 += rows
      if i == 0:
        if drain_g is not None:
          chunk_scatter_desc(drain_g, drain_b).wait()
        if issue_g is not None:
          gather_desc(issue_g, issue_b).start()

  gather_desc(0, 0).start()
  pos_tail = pltpu.async_copy(
      pos_hbm.at[pl.ds(POS_HEAD, SEQ - POS_HEAD)],
      pos_v.at[pl.ds(POS_HEAD, SEQ - POS_HEAD)], psem)
  gather_desc(1, 1).start()
  pos_head.wait()
  gather_desc(0, 0).wait()
  r0 = 0
  for i, rows in enumerate(PART_SIZES):
    if r0 + rows > POS_HEAD and r0 <= POS_HEAD:
      pos_tail.wait()
    add_part(0, r0, rows)
    part_scatter(0, 0, r0, rows)
    r0 += rows
    if i == 0:
      gather_desc(2, 2).start()
  process_chunk(1, 1, issue_g=3, issue_b=0, drain_g=0, drain_b=0)

  def triple_body(q, carry):
    for j in range(3):
      g = 3 * q + 2 + j
      b = (2 + j) % 3
      recycle_b = (1 + j) % 3
      process_chunk(g, b, issue_g=g + 2, issue_b=recycle_b,
                    drain_g=g - 1, drain_b=recycle_b)
    return carry
  lax.fori_loop(0, (N_CHUNKS - 5) // 3, triple_body, 0)

  process_chunk(N_CHUNKS - 3, (N_CHUNKS - 3) % 3,
                issue_g=N_CHUNKS - 1, issue_b=(N_CHUNKS - 1) % 3,
                drain_g=N_CHUNKS - 4, drain_b=(N_CHUNKS - 4) % 3)
  process_chunk(N_CHUNKS - 2, (N_CHUNKS - 2) % 3)
  process_chunk(N_CHUNKS - 1, (N_CHUNKS - 1) % 3)
  for g in (N_CHUNKS - 3, N_CHUNKS - 2, N_CHUNKS - 1):
    chunk_scatter_desc(g, g % 3).wait()


@jax.jit
def _run(idx_flat, table, pos):
  kern = pl.kernel(
      _sc_body,
      out_type=jax.ShapeDtypeStruct((B_TOTAL, DIM), jnp.float32),
      mesh=plsc.VectorSubcoreMesh(
          core_axis_name="c", subcore_axis_name="s",
          num_cores=NC, num_subcores=NS),
      scratch_types=[
          pltpu.VMEM((B_PER_W,), jnp.int32),
          pltpu.VMEM((SEQ, DIM), jnp.float32),
          pltpu.VMEM((CHUNK, DIM), jnp.float32),
          pltpu.VMEM((CHUNK, DIM), jnp.float32),
          pltpu.VMEM((CHUNK, DIM), jnp.float32),
          pltpu.SemaphoreType.DMA,
          pltpu.SemaphoreType.DMA,
          pltpu.SemaphoreType.DMA,
          pltpu.SemaphoreType.DMA,
          pltpu.SemaphoreType.DMA,
          pltpu.SemaphoreType.DMA,
          pltpu.SemaphoreType.DMA,
      ],
  )
  return kern(idx_flat, table, pos)


def kernel(inputs, input_embedding_matrix, position_embedding_matrix):
  idx_flat = inputs.reshape(B_TOTAL)
  out = _run(idx_flat, input_embedding_matrix, position_embedding_matrix)
  return out.reshape(BATCH, SEQ, DIM)
